# baseline (speedup 1.0000x reference)
import jax
import jax.numpy as jnp
from jax.experimental import pallas as pl
from jax.experimental.pallas import tpu as pltpu

_BLOCK = 2048


def _fused_body(x_ref, w1_ref, thr_ref, w2_ref, o_ref):
    u = jnp.dot(x_ref[:], w1_ref[:], preferred_element_type=jnp.float32)
    h = jnp.tanh(u - thr_ref[:])
    o = jnp.tanh(jnp.dot(h, w2_ref[:], preferred_element_type=jnp.float32) - 0.5)
    o_ref[:] = o


def kernel(x, W1, thr_h, W2):
    batch, in_size = x.shape
    hidden = W1.shape[1]
    out_size = W2.shape[1]
    thr2d = thr_h.reshape(1, hidden)

    grid = (batch // _BLOCK,)
    return pl.pallas_call(
        _fused_body,
        grid=grid,
        in_specs=[
            pl.BlockSpec((_BLOCK, in_size), lambda i: (i, 0)),
            pl.BlockSpec((in_size, hidden), lambda i: (0, 0)),
            pl.BlockSpec((1, hidden), lambda i: (0, 0)),
            pl.BlockSpec((hidden, out_size), lambda i: (0, 0)),
        ],
        out_specs=pl.BlockSpec((_BLOCK, out_size), lambda i: (i, 0)),
        out_shape=jax.ShapeDtypeStruct((batch, out_size), jnp.float32),
        compiler_params=pltpu.CompilerParams(
            dimension_semantics=("arbitrary",),
        ),
    )(x, W1, thr2d, W2)
